# initial kernel scaffold (unmeasured)
import jax
import jax.numpy as jnp
from jax import lax
from jax.experimental import pallas as pl
from jax.experimental.pallas import tpu as pltpu

N_DEV = 4
SQ = 1024
SKV = 1024
HQ = 8
DH = 128
D = HQ * DH
BLK = 64
QC = 256
SCALE = 0.08838834764831843


def _comm_body(kv_ref, out_ref, send_sems, recv_sem):
    my = lax.axis_index("i")

    def mk(src, dst, ssem, tgt):
        return pltpu.make_async_remote_copy(
            src_ref=src,
            dst_ref=dst,
            send_sem=ssem,
            recv_sem=recv_sem,
            device_id=(tgt,),
            device_id_type=pl.DeviceIdType.MESH,
        )

    @pl.when(my == 0)
    def _():
        out_ref[...] = kv_ref[...]
        c1 = mk(kv_ref, out_ref, send_sems.at[0], 1)
        c3 = mk(kv_ref, out_ref, send_sems.at[1], 3)
        c1.start()
        c3.start()
        c1.wait_send()
        c3.wait_send()

    @pl.when(my == 1)
    def _():
        r = mk(kv_ref, out_ref, send_sems.at[0], 0)
        r.wait_recv()
        f = mk(out_ref, out_ref, send_sems.at[0], 2)
        f.start()
        f.wait_send()

    @pl.when(jnp.logical_or(my == 2, my == 3))
    def _():
        r = mk(kv_ref, out_ref, send_sems.at[0], 0)
        r.wait_recv()


def _attn_body(x_ref, wq_ref, k_ref, v_ref, wo_ref, out_ref):
    qc = pl.program_id(0)
    h = pl.program_id(1)

    q = jnp.dot(x_ref[:, :], wq_ref[:, :], preferred_element_type=jnp.float32)
    k = k_ref[0, :, 0, :]
    v = v_ref[0, :, 0, :]

    s = lax.dot_general(
        q, k, (((1,), (1,)), ((), ())), preferred_element_type=jnp.float32
    ) * SCALE

    qb = (qc * QC + lax.broadcasted_iota(jnp.int32, (QC, SKV), 0)) // BLK
    kb = lax.broadcasted_iota(jnp.int32, (QC, SKV), 1) // BLK
    s = jnp.where(kb <= qb, s, -1e9)

    m = jnp.max(s, axis=1, keepdims=True)
    w = jnp.exp(s - m)
    p = w / jnp.sum(w, axis=1, keepdims=True)

    ctx = jnp.dot(p, v, preferred_element_type=jnp.float32)
    contrib = jnp.dot(ctx, wo_ref[:, :], preferred_element_type=jnp.float32)

    @pl.when(h == 0)
    def _():
        out_ref[...] = jnp.zeros_like(out_ref)

    out_ref[...] += contrib


def kernel(x, Wq, K_ext, V_ext, Wo):
    xm = x[0]
    kv = jnp.stack([K_ext[0], V_ext[0]])

    kv0 = pl.pallas_call(
        _comm_body,
        out_shape=jax.ShapeDtypeStruct((2, SKV, HQ, DH), jnp.float32),
        in_specs=[pl.BlockSpec(memory_space=pltpu.VMEM)],
        out_specs=pl.BlockSpec(memory_space=pltpu.VMEM),
        scratch_shapes=[
            pltpu.SemaphoreType.DMA((2,)),
            pltpu.SemaphoreType.DMA,
        ],
        compiler_params=pltpu.CompilerParams(collective_id=0),
    )(kv)

    out = pl.pallas_call(
        _attn_body,
        grid=(SQ // QC, HQ),
        out_shape=jax.ShapeDtypeStruct((SQ, D), jnp.float32),
        in_specs=[
            pl.BlockSpec((QC, D), lambda qc, h: (qc, 0)),
            pl.BlockSpec((D, DH), lambda qc, h: (0, h)),
            pl.BlockSpec((1, SKV, 1, DH), lambda qc, h: (0, 0, h, 0)),
            pl.BlockSpec((1, SKV, 1, DH), lambda qc, h: (1, 0, h, 0)),
            pl.BlockSpec((DH, D), lambda qc, h: (h, 0)),
        ],
        out_specs=pl.BlockSpec((QC, D), lambda qc, h: (qc, 0)),
    )(xm, Wq, kv0, kv0, Wo)

    return out.reshape(1, SQ, D)


# baseline (device time: 254245 ns/iter reference)
import jax
import jax.numpy as jnp
from jax import lax
from jax.experimental import pallas as pl
from jax.experimental.pallas import tpu as pltpu

N_DEV = 4
SQ = 1024
SKV = 1024
HQ = 8
DH = 128
D = HQ * DH
BLK = 64
QC = 256
SCALE = 0.08838834764831843


def _comm_body(kv_ref, out_ref, send_sems, recv_sem):
    my = lax.axis_index("i")

    def mk(src, dst, ssem, tgt):
        return pltpu.make_async_remote_copy(
            src_ref=src,
            dst_ref=dst,
            send_sem=ssem,
            recv_sem=recv_sem,
            device_id=(tgt,),
            device_id_type=pl.DeviceIdType.MESH,
        )

    @pl.when(my == 0)
    def _():
        out_ref[...] = kv_ref[...]
        c1 = mk(kv_ref, out_ref, send_sems.at[0], 1)
        c3 = mk(kv_ref, out_ref, send_sems.at[1], 3)
        c1.start()
        c3.start()
        c1.wait_send()
        c3.wait_send()

    @pl.when(my == 1)
    def _():
        r = mk(kv_ref, out_ref, send_sems.at[0], 0)
        r.wait_recv()
        f = mk(out_ref, out_ref, send_sems.at[0], 2)
        f.start()
        f.wait_send()

    @pl.when(jnp.logical_or(my == 2, my == 3))
    def _():
        r = mk(kv_ref, out_ref, send_sems.at[0], 0)
        r.wait_recv()


def _attn_body(x_ref, wq_ref, k_ref, v_ref, wo_ref, out_ref):
    qc = pl.program_id(0)
    h = pl.program_id(1)

    q = jnp.dot(x_ref[:, :], wq_ref[:, :], preferred_element_type=jnp.float32)
    k = k_ref[0, 0, :, :]
    v = v_ref[0, 0, :, :]

    s = lax.dot_general(
        q, k, (((1,), (1,)), ((), ())), preferred_element_type=jnp.float32
    ) * SCALE

    qb = (qc * QC + lax.broadcasted_iota(jnp.int32, (QC, SKV), 0)) // BLK
    kb = lax.broadcasted_iota(jnp.int32, (QC, SKV), 1) // BLK
    s = jnp.where(kb <= qb, s, -1e9)

    m = jnp.max(s, axis=1, keepdims=True)
    w = jnp.exp(s - m)
    p = w / jnp.sum(w, axis=1, keepdims=True)

    ctx = jnp.dot(p, v, preferred_element_type=jnp.float32)
    contrib = jnp.dot(ctx, wo_ref[:, :], preferred_element_type=jnp.float32)

    @pl.when(h == 0)
    def _():
        out_ref[...] = jnp.zeros_like(out_ref)

    out_ref[...] += contrib


def kernel(x, Wq, K_ext, V_ext, Wo):
    xm = x[0]
    kv = jnp.stack(
        [K_ext[0].transpose(1, 0, 2), V_ext[0].transpose(1, 0, 2)]
    )

    kv0 = pl.pallas_call(
        _comm_body,
        out_shape=jax.ShapeDtypeStruct((2, HQ, SKV, DH), jnp.float32),
        in_specs=[pl.BlockSpec(memory_space=pltpu.VMEM)],
        out_specs=pl.BlockSpec(memory_space=pltpu.VMEM),
        scratch_shapes=[
            pltpu.SemaphoreType.DMA((2,)),
            pltpu.SemaphoreType.DMA,
        ],
    )(kv)

    out = pl.pallas_call(
        _attn_body,
        grid=(SQ // QC, HQ),
        out_shape=jax.ShapeDtypeStruct((SQ, D), jnp.float32),
        in_specs=[
            pl.BlockSpec((QC, D), lambda qc, h: (qc, 0)),
            pl.BlockSpec((D, DH), lambda qc, h: (0, h)),
            pl.BlockSpec((1, 1, SKV, DH), lambda qc, h: (0, h, 0, 0)),
            pl.BlockSpec((1, 1, SKV, DH), lambda qc, h: (1, h, 0, 0)),
            pl.BlockSpec((DH, D), lambda qc, h: (h, 0)),
        ],
        out_specs=pl.BlockSpec((QC, D), lambda qc, h: (qc, 0)),
    )(xm, Wq, kv0, kv0, Wo)

    return out.reshape(1, SQ, D)


# device time: 74260 ns/iter; 3.4237x vs baseline; 3.4237x over previous
import jax
import jax.numpy as jnp
from jax import lax
from jax.experimental import pallas as pl
from jax.experimental.pallas import tpu as pltpu

N_DEV = 4
SQ = 1024
SKV = 1024
HQ = 8
DH = 128
D = HQ * DH
BLK = 64
SCALE = 0.08838834764831843


def _body(x_ref, wq_ref, kv_ref, wo_ref, out_ref,
          comm_ref, ctx_ref, send_sems, recv_sems):
    my = lax.axis_index("i")

    def mk(h, ssem, tgt):
        return pltpu.make_async_remote_copy(
            src_ref=comm_ref.at[h],
            dst_ref=comm_ref.at[h],
            send_sem=ssem,
            recv_sem=recv_sems.at[h],
            device_id=(tgt,),
            device_id_type=pl.DeviceIdType.MESH,
        )

    to1 = [mk(h, send_sems.at[0, h], 1) for h in range(HQ)]
    to3 = [mk(h, send_sems.at[1, h], 3) for h in range(HQ)]
    fwd = [mk(h, send_sems.at[0, h], 2) for h in range(HQ)]

    @pl.when(my == 0)
    def _():
        comm_ref[...] = kv_ref[...]
        for h in range(HQ):
            to1[h].start()
            to3[h].start()

    q_all = jnp.dot(x_ref[...], wq_ref[...],
                    preferred_element_type=jnp.float32).astype(jnp.bfloat16)

    qb = lax.broadcasted_iota(jnp.int32, (SQ, SKV), 0) // BLK
    kb = lax.broadcasted_iota(jnp.int32, (SQ, SKV), 1) // BLK
    mask = kb <= qb

    for h in range(HQ):
        @pl.when(my != 0)
        def _(h=h):
            to1[h].wait_recv()

        @pl.when(my == 1)
        def _(h=h):
            fwd[h].start()

        k = comm_ref[h, 0]
        v = comm_ref[h, 1]
        qh = q_all[:, h * DH:(h + 1) * DH]
        s = lax.dot_general(
            qh, k, (((1,), (1,)), ((), ())),
            preferred_element_type=jnp.float32,
        ) * SCALE
        s = jnp.where(mask, s, -1e9)
        m = jnp.max(s, axis=1, keepdims=True)
        w = jnp.exp(s - m)
        p = (w / jnp.sum(w, axis=1, keepdims=True)).astype(jnp.bfloat16)
        ctx = jnp.dot(p, v, preferred_element_type=jnp.float32)
        ctx_ref[:, h * DH:(h + 1) * DH] = ctx.astype(jnp.bfloat16)

    out_ref[...] = jnp.dot(ctx_ref[...], wo_ref[...],
                           preferred_element_type=jnp.float32)

    @pl.when(my == 0)
    def _():
        for h in range(HQ):
            to1[h].wait_send()
            to3[h].wait_send()

    @pl.when(my == 1)
    def _():
        for h in range(HQ):
            fwd[h].wait_send()


def kernel(x, Wq, K_ext, V_ext, Wo):
    bf16 = jnp.bfloat16
    xb = x[0].astype(bf16)
    wqb = Wq.astype(bf16)
    wob = Wo.astype(bf16)
    kvb = jnp.stack(
        [K_ext[0].astype(bf16).transpose(1, 0, 2),
         V_ext[0].astype(bf16).transpose(1, 0, 2)],
        axis=1,
    )

    out = pl.pallas_call(
        _body,
        out_shape=jax.ShapeDtypeStruct((SQ, D), jnp.float32),
        in_specs=[pl.BlockSpec(memory_space=pltpu.VMEM)] * 4,
        out_specs=pl.BlockSpec(memory_space=pltpu.VMEM),
        scratch_shapes=[
            pltpu.VMEM((HQ, 2, SKV, DH), bf16),
            pltpu.VMEM((SQ, D), bf16),
            pltpu.SemaphoreType.DMA((2, HQ)),
            pltpu.SemaphoreType.DMA((HQ,)),
        ],
    )(xb, wqb, kvb, wob)

    return out.reshape(1, SQ, D)
